# Initial kernel scaffold; baseline (speedup 1.0000x reference)
import jax
import jax.numpy as jnp
from jax.experimental import pallas as pl


def kernel(x, edge_index, W, b, a):
    raise NotImplementedError("write your pallas kernel here")


# speedup vs baseline: 14.3055x; 14.3055x over previous
import functools

import jax
import jax.numpy as jnp
from jax import lax
from jax.experimental import pallas as pl
from jax.experimental.pallas import tpu as pltpu
from jax.experimental.pallas import tpu_sc as plsc

N = 10000
E = 160000
D = 256
DH = 128
NPAD = 10240
EPAD = 163840
CH = 128
NC = 2
NS = 16
RPT = NPAD // NS


def _sc_degree(dst2, zerosu, ones128):
    n_rows = EPAD // CH // (NC * NS)

    mesh = plsc.VectorSubcoreMesh(core_axis_name="c", subcore_axis_name="s")

    @functools.partial(
        pl.kernel,
        mesh=mesh,
        out_type=jax.ShapeDtypeStruct((NC * NPAD, DH), jnp.float32),
        scratch_types=[
            pltpu.VMEM((n_rows, CH), jnp.int32),
            pltpu.VMEM((CH, DH), jnp.float32),
            pltpu.VMEM_SHARED((NPAD, DH), jnp.float32),
        ],
    )
    def k(dst_hbm, z_hbm, ones_hbm, out_hbm, didx, ones_v, acc):
        c = lax.axis_index("c")
        s = lax.axis_index("s")
        wid = s * NC + c
        r0 = s * RPT
        pltpu.sync_copy(z_hbm.at[pl.ds(r0, RPT)], acc.at[pl.ds(r0, RPT)])
        pltpu.sync_copy(ones_hbm, ones_v)
        pltpu.sync_copy(dst_hbm.at[pl.ds(wid * n_rows, n_rows)], didx)
        plsc.subcore_barrier()

        def chunk(kk, carry):
            pltpu.sync_copy(ones_v, acc.at[didx.at[kk]], add=True)
            return carry

        lax.fori_loop(0, n_rows, chunk, 0)
        plsc.subcore_barrier()
        pltpu.sync_copy(acc.at[pl.ds(r0, RPT)],
                        out_hbm.at[pl.ds(c * NPAD + r0, RPT)])

    return k(dst2, zerosu, ones128)


def _tc_u(x_pad, W, p):
    R = 512
    G = NPAD // R

    def body(x_ref, w_ref, p0_ref, p1_ref, u0_ref, u1_ref):
        deg = 1.0 + p0_ref[:, 0:1] + p1_ref[:, 0:1]
        dinv = lax.rsqrt(deg)
        h = jnp.dot(x_ref[...], w_ref[...], preferred_element_type=jnp.float32)
        u = dinv * h
        u0_ref[...] = u[:, :DH]
        u1_ref[...] = u[:, DH:]

    return pl.pallas_call(
        body,
        grid=(G,),
        in_specs=[
            pl.BlockSpec((R, D), lambda i: (i, 0)),
            pl.BlockSpec((D, D), lambda i: (0, 0)),
            pl.BlockSpec((R, DH), lambda i: (i, 0)),
            pl.BlockSpec((R, DH), lambda i: (i + NPAD // R, 0)),
        ],
        out_specs=[
            pl.BlockSpec((R, DH), lambda i: (i, 0)),
            pl.BlockSpec((R, DH), lambda i: (i, 0)),
        ],
        out_shape=[jax.ShapeDtypeStruct((NPAD, DH), jnp.float32)] * 2,
    )(x_pad, W, p, p)


def _sc_agg(u0, u1, src2, dst2, zerosu):
    n_rows = EPAD // CH // NS

    mesh = plsc.VectorSubcoreMesh(core_axis_name="c", subcore_axis_name="s")

    @functools.partial(
        pl.kernel,
        mesh=mesh,
        out_type=jax.ShapeDtypeStruct((NC * NPAD, DH), jnp.float32),
        scratch_types=[
            pltpu.VMEM((n_rows, CH), jnp.int32),
            pltpu.VMEM((n_rows, CH), jnp.int32),
            pltpu.VMEM((CH, DH), jnp.float32),
            pltpu.VMEM_SHARED((NPAD, DH), jnp.float32),
            pltpu.SemaphoreType.DMA,
        ],
    )
    def k(u0_hbm, u1_hbm, src_hbm, dst_hbm, z_hbm, out_hbm,
          sidx, didx, rows, acc, sem):
        c = lax.axis_index("c")
        s = lax.axis_index("s")
        r0 = s * RPT
        pltpu.sync_copy(z_hbm.at[pl.ds(r0, RPT)], acc.at[pl.ds(r0, RPT)])
        pltpu.sync_copy(src_hbm.at[pl.ds(s * n_rows, n_rows)], sidx)
        pltpu.sync_copy(dst_hbm.at[pl.ds(s * n_rows, n_rows)], didx)
        plsc.subcore_barrier()

        def chunk(kk, carry):
            @pl.when(c == 0)
            def _():
                pltpu.async_copy(u0_hbm.at[sidx.at[kk]], rows, sem).wait()

            @pl.when(c == 1)
            def _():
                pltpu.async_copy(u1_hbm.at[sidx.at[kk]], rows, sem).wait()

            pltpu.sync_copy(rows, acc.at[didx.at[kk]], add=True)
            return carry

        lax.fori_loop(0, n_rows, chunk, 0)
        plsc.subcore_barrier()
        pltpu.sync_copy(acc.at[pl.ds(r0, RPT)],
                        out_hbm.at[pl.ds(c * NPAD + r0, RPT)])

    return k(u0, u1, src2, dst2, zerosu)


def _tc_final(p, agg, u0, u1, b2, a2):
    R = 512
    G = NPAD // R

    def body(p0_ref, p1_ref, a0_ref, a1_ref, u0_ref, u1_ref, b_ref, al_ref,
             o_ref):
        deg = 1.0 + p0_ref[:, 0:1] + p1_ref[:, 0:1]
        dinv = lax.rsqrt(deg)
        al = al_ref[0, 0]
        v0 = dinv * (a0_ref[...] + u0_ref[...]) + b_ref[:, :DH]
        v1 = dinv * (a1_ref[...] + u1_ref[...]) + b_ref[:, DH:]
        o_ref[:, :DH] = jnp.where(v0 >= 0, v0, al * v0)
        o_ref[:, DH:] = jnp.where(v1 >= 0, v1, al * v1)

    return pl.pallas_call(
        body,
        grid=(G,),
        in_specs=[
            pl.BlockSpec((R, DH), lambda i: (i, 0)),
            pl.BlockSpec((R, DH), lambda i: (i + NPAD // R, 0)),
            pl.BlockSpec((R, DH), lambda i: (i, 0)),
            pl.BlockSpec((R, DH), lambda i: (i + NPAD // R, 0)),
            pl.BlockSpec((R, DH), lambda i: (i, 0)),
            pl.BlockSpec((R, DH), lambda i: (i, 0)),
            pl.BlockSpec((1, D), lambda i: (0, 0)),
            pl.BlockSpec((1, 1), lambda i: (0, 0)),
        ],
        out_specs=pl.BlockSpec((R, D), lambda i: (i, 0)),
        out_shape=jax.ShapeDtypeStruct((NPAD, D), jnp.float32),
    )(p, p, agg, agg, u0, u1, b2, a2)


def kernel(x, edge_index, W, b, a):
    src = edge_index[0]
    dst = edge_index[1]
    pad = N + (jnp.arange(EPAD - E, dtype=jnp.int32) % (NPAD - N))
    src2 = jnp.concatenate([src, pad]).reshape(EPAD // CH, CH)
    dst2 = jnp.concatenate([dst, pad]).reshape(EPAD // CH, CH)
    x_pad = jnp.pad(x, ((0, NPAD - N), (0, 0)))
    ones128 = jnp.ones((CH, DH), jnp.float32)
    zerosu = jnp.zeros((NPAD, DH), jnp.float32)

    p = _sc_degree(dst2, zerosu, ones128)
    u0, u1 = _tc_u(x_pad, W, p)
    agg = _sc_agg(u0, u1, src2, dst2, zerosu)
    out = _tc_final(p, agg, u0, u1, b.reshape(1, D), a.reshape(1, 1))
    return out[:N]


# speedup vs baseline: 18.8500x; 1.3177x over previous
import functools

import jax
import jax.numpy as jnp
from jax import lax
from jax.experimental import pallas as pl
from jax.experimental.pallas import tpu as pltpu
from jax.experimental.pallas import tpu_sc as plsc

N = 10000
E = 160000
D = 256
DH = 128
NPAD = 10240
EPAD = 163840
CH = 128
NC = 2
NS = 16
RPT = NPAD // NS


def _sc_degree(dst2, zeros, ones128):
    n_rows = EPAD // CH // (NC * NS)

    mesh = plsc.VectorSubcoreMesh(core_axis_name="c", subcore_axis_name="s")

    @functools.partial(
        pl.kernel,
        mesh=mesh,
        out_type=jax.ShapeDtypeStruct((NC * NPAD, DH), jnp.float32),
        scratch_types=[
            pltpu.VMEM((n_rows, CH), jnp.int32),
            pltpu.VMEM((CH, DH), jnp.float32),
            pltpu.VMEM_SHARED((NPAD, DH), jnp.float32),
        ],
    )
    def k(dst_hbm, z_hbm, ones_hbm, out_hbm, didx, ones_v, acc):
        c = lax.axis_index("c")
        s = lax.axis_index("s")
        wid = s * NC + c
        r0 = s * RPT
        pltpu.sync_copy(z_hbm, acc.at[pl.ds(r0, RPT)])
        pltpu.sync_copy(ones_hbm, ones_v)
        pltpu.sync_copy(dst_hbm.at[pl.ds(wid * n_rows, n_rows)], didx)
        plsc.subcore_barrier()

        def chunk(kk, carry):
            pltpu.sync_copy(ones_v, acc.at[didx.at[kk]], add=True)
            return carry

        lax.fori_loop(0, n_rows, chunk, 0)
        plsc.subcore_barrier()
        pltpu.sync_copy(acc.at[pl.ds(r0, RPT)],
                        out_hbm.at[pl.ds(c * NPAD + r0, RPT)])

    return k(dst2, zeros, ones128)


def _tc_u(x, W, p3):
    R = 400
    G = N // R

    def body(x_ref, w_ref, p0_ref, p1_ref, u0_ref, u1_ref):
        deg = 1.0 + p0_ref[0, :, 0:1] + p1_ref[0, :, 0:1]
        dinv = lax.rsqrt(deg)
        h = jnp.dot(x_ref[...], w_ref[...], preferred_element_type=jnp.float32)
        u = dinv * h
        u0_ref[...] = u[:, :DH]
        u1_ref[...] = u[:, DH:]

    return pl.pallas_call(
        body,
        grid=(G,),
        in_specs=[
            pl.BlockSpec((R, D), lambda i: (i, 0)),
            pl.BlockSpec((D, D), lambda i: (0, 0)),
            pl.BlockSpec((1, R, DH), lambda i: (0, i, 0)),
            pl.BlockSpec((1, R, DH), lambda i: (1, i, 0)),
        ],
        out_specs=[
            pl.BlockSpec((R, DH), lambda i: (i, 0)),
            pl.BlockSpec((R, DH), lambda i: (i, 0)),
        ],
        out_shape=[jax.ShapeDtypeStruct((N, DH), jnp.float32)] * 2,
    )(x, W, p3, p3)


def _sc_agg(u0, u1, src2, dst2, zeros):
    n_rows = EPAD // CH // NS
    half = n_rows // 2

    mesh = plsc.VectorSubcoreMesh(core_axis_name="c", subcore_axis_name="s")

    @functools.partial(
        pl.kernel,
        mesh=mesh,
        out_type=jax.ShapeDtypeStruct((NC * NPAD, DH), jnp.float32),
        scratch_types=[
            pltpu.VMEM((half, CH), jnp.int32),
            pltpu.VMEM((half, CH), jnp.int32),
            pltpu.VMEM((CH, DH), jnp.float32),
            pltpu.VMEM((CH, DH), jnp.float32),
            pltpu.VMEM_SHARED((NPAD, DH), jnp.float32),
            pltpu.SemaphoreType.DMA,
            pltpu.SemaphoreType.DMA,
        ],
    )
    def k(u0_hbm, u1_hbm, src_hbm, dst_hbm, z_hbm, out_hbm,
          sidx, didx, rows0, rows1, acc, sem0, sem1):
        c = lax.axis_index("c")
        s = lax.axis_index("s")
        r0 = s * RPT
        pltpu.sync_copy(z_hbm, acc.at[pl.ds(r0, RPT)])
        plsc.subcore_barrier()

        def run(u_hbm):
            for h in range(2):
                base = s * n_rows + h * half
                pltpu.sync_copy(src_hbm.at[pl.ds(base, half)], sidx)
                pltpu.sync_copy(dst_hbm.at[pl.ds(base, half)], didx)
                pltpu.async_copy(u_hbm.at[sidx.at[0]], rows0, sem0)
                pltpu.async_copy(u_hbm.at[sidx.at[1]], rows1, sem1)

                def body(j, carry):
                    kk = 2 * j
                    pltpu.make_async_copy(u_hbm.at[sidx.at[kk]], rows0,
                                          sem0).wait()
                    pltpu.sync_copy(rows0, acc.at[didx.at[kk]], add=True)

                    @pl.when(j + 1 < half // 2)
                    def _():
                        pltpu.async_copy(u_hbm.at[sidx.at[kk + 2]], rows0,
                                         sem0)

                    pltpu.make_async_copy(u_hbm.at[sidx.at[kk + 1]], rows1,
                                          sem1).wait()
                    pltpu.sync_copy(rows1, acc.at[didx.at[kk + 1]], add=True)

                    @pl.when(j + 1 < half // 2)
                    def _():
                        pltpu.async_copy(u_hbm.at[sidx.at[kk + 3]], rows1,
                                         sem1)

                    return carry

                lax.fori_loop(0, half // 2, body, 0)

        @pl.when(c == 0)
        def _():
            run(u0_hbm)

        @pl.when(c == 1)
        def _():
            run(u1_hbm)

        plsc.subcore_barrier()
        pltpu.sync_copy(acc.at[pl.ds(r0, RPT)],
                        out_hbm.at[pl.ds(c * NPAD + r0, RPT)])

    return k(u0, u1, src2, dst2, zeros)


def _tc_final(p3, agg3, u0, u1, b2, a2):
    R = 400
    G = N // R

    def body(p0_ref, p1_ref, a0_ref, a1_ref, u0_ref, u1_ref, b_ref, al_ref,
             o_ref):
        deg = 1.0 + p0_ref[0, :, 0:1] + p1_ref[0, :, 0:1]
        dinv = lax.rsqrt(deg)
        al = al_ref[0, 0]
        v0 = dinv * (a0_ref[0] + u0_ref[...]) + b_ref[:, :DH]
        v1 = dinv * (a1_ref[0] + u1_ref[...]) + b_ref[:, DH:]
        o_ref[:, :DH] = jnp.where(v0 >= 0, v0, al * v0)
        o_ref[:, DH:] = jnp.where(v1 >= 0, v1, al * v1)

    return pl.pallas_call(
        body,
        grid=(G,),
        in_specs=[
            pl.BlockSpec((1, R, DH), lambda i: (0, i, 0)),
            pl.BlockSpec((1, R, DH), lambda i: (1, i, 0)),
            pl.BlockSpec((1, R, DH), lambda i: (0, i, 0)),
            pl.BlockSpec((1, R, DH), lambda i: (1, i, 0)),
            pl.BlockSpec((R, DH), lambda i: (i, 0)),
            pl.BlockSpec((R, DH), lambda i: (i, 0)),
            pl.BlockSpec((1, D), lambda i: (0, 0)),
            pl.BlockSpec((1, 1), lambda i: (0, 0)),
        ],
        out_specs=pl.BlockSpec((R, D), lambda i: (i, 0)),
        out_shape=jax.ShapeDtypeStruct((N, D), jnp.float32),
    )(p3, p3, agg3, agg3, u0, u1, b2, a2)


def kernel(x, edge_index, W, b, a):
    src = edge_index[0]
    dst = edge_index[1]
    npd = EPAD - E
    pad_src = jnp.arange(npd, dtype=jnp.int32) % N
    pad_dst = N + (jnp.arange(npd, dtype=jnp.int32) % (NPAD - N))
    src2 = jnp.concatenate([src, pad_src]).reshape(EPAD // CH, CH)
    dst2 = jnp.concatenate([dst, pad_dst]).reshape(EPAD // CH, CH)
    ones128 = jnp.ones((CH, DH), jnp.float32)
    zeros = jnp.zeros((RPT, DH), jnp.float32)

    p = _sc_degree(dst2, zeros, ones128)
    p3 = p.reshape(NC, NPAD, DH)
    u0, u1 = _tc_u(x, W, p3)
    agg = _sc_agg(u0, u1, src2, dst2, zeros)
    agg3 = agg.reshape(NC, NPAD, DH)
    return _tc_final(p3, agg3, u0, u1, b.reshape(1, D), a.reshape(1, 1))
